# initial kernel scaffold (unmeasured)
import jax
import jax.numpy as jnp
from jax import lax
from jax.experimental import pallas as pl
from jax.experimental.pallas import tpu as pltpu

N_DEV = 4
B_PER = 2
SQ = 512
SKV = 512
HG = 32
H_PER = 8
DH = 64
D_MODEL = 768
SCALE = 0.125


def kernel(x, Wq, K_ext, V_ext, Wo):
    my = lax.axis_index("i")

    k_my = lax.dynamic_slice(K_ext, (my * B_PER, 0, 0, 0), (B_PER, SKV, HG, DH))
    v_my = lax.dynamic_slice(V_ext, (my * B_PER, 0, 0, 0), (B_PER, SKV, HG, DH))
    k_r = k_my.reshape(B_PER, SKV, N_DEV, H_PER, DH).transpose(2, 0, 3, 1, 4)
    v_r = v_my.reshape(B_PER, SKV, N_DEV, H_PER, DH).transpose(2, 0, 3, 1, 4)

    def body(x_ref, wq_ref, k_ref, v_ref, wo_ref, out_ref,
             wq_comm, wo_comm, ctx_ref, send_q, recv_q, send_o, recv_o):
        my_pos = lax.axis_index("i")
        left = lax.rem(my_pos + N_DEV - 1, N_DEV)
        right = lax.rem(my_pos + 1, N_DEV)

        barrier_sem = pltpu.get_barrier_semaphore()
        for nbr in (left, right):
            pl.semaphore_signal(
                barrier_sem, inc=1,
                device_id=(nbr,), device_id_type=pl.DeviceIdType.MESH,
            )
        pl.semaphore_wait(barrier_sem, 2)

        qi = lax.broadcasted_iota(jnp.int32, (SQ, SKV), 0)
        ki = lax.broadcasted_iota(jnp.int32, (SQ, SKV), 1)
        mask = (jnp.abs(qi - ki) <= 128) | (ki < 32) | (qi < 32)

        def compute_chunk(wq_c, wo_c, origin, first):
            for b in range(B_PER):
                q = lax.dot_general(
                    x_ref[b], wq_c, (((1,), (0,)), ((), ())),
                    preferred_element_type=jnp.float32,
                )
                for h in range(H_PER):
                    qh = q[:, h * DH:(h + 1) * DH]
                    kh = k_ref[origin, b, h]
                    s = lax.dot_general(
                        qh, kh, (((1,), (1,)), ((), ())),
                        preferred_element_type=jnp.float32,
                    ) * SCALE
                    s = jnp.where(mask, s, -1e9)
                    s = s - jnp.max(s, axis=-1, keepdims=True)
                    w = jnp.exp(s)
                    w = w / jnp.sum(w, axis=-1, keepdims=True)
                    vh = v_ref[origin, b, h]
                    ctx_ref[:, h * DH:(h + 1) * DH] = lax.dot_general(
                        w, vh, (((1,), (0,)), ((), ())),
                        preferred_element_type=jnp.float32,
                    )
                partial = lax.dot_general(
                    ctx_ref[...], wo_c, (((1,), (0,)), ((), ())),
                    preferred_element_type=jnp.float32,
                )
                if first:
                    out_ref[b] = partial
                else:
                    out_ref[b] = out_ref[b] + partial

        for h in range(N_DEV - 1):
            src_q = wq_ref if h == 0 else wq_comm.at[h - 1]
            src_o = wo_ref if h == 0 else wo_comm.at[h - 1]
            rdma_q = pltpu.make_async_remote_copy(
                src_ref=src_q, dst_ref=wq_comm.at[h],
                send_sem=send_q.at[h], recv_sem=recv_q.at[h],
                device_id=(right,), device_id_type=pl.DeviceIdType.MESH,
            )
            rdma_o = pltpu.make_async_remote_copy(
                src_ref=src_o, dst_ref=wo_comm.at[h],
                send_sem=send_o.at[h], recv_sem=recv_o.at[h],
                device_id=(right,), device_id_type=pl.DeviceIdType.MESH,
            )
            rdma_q.start()
            rdma_o.start()
            origin = lax.rem(my_pos - h + N_DEV, N_DEV)
            if h == 0:
                compute_chunk(wq_ref[...], wo_ref[...], origin, first=True)
            else:
                compute_chunk(wq_comm[h - 1], wo_comm[h - 1], origin, first=False)
            rdma_q.wait()
            rdma_o.wait()

        origin = lax.rem(my_pos - (N_DEV - 1) + N_DEV, N_DEV)
        compute_chunk(wq_comm[N_DEV - 2], wo_comm[N_DEV - 2], origin, first=False)

    return pl.pallas_call(
        body,
        out_shape=jax.ShapeDtypeStruct((B_PER, SQ, D_MODEL), jnp.float32),
        in_specs=[
            pl.BlockSpec(memory_space=pltpu.VMEM),
            pl.BlockSpec(memory_space=pltpu.VMEM),
            pl.BlockSpec(memory_space=pltpu.VMEM),
            pl.BlockSpec(memory_space=pltpu.VMEM),
            pl.BlockSpec(memory_space=pltpu.VMEM),
        ],
        out_specs=pl.BlockSpec(memory_space=pltpu.VMEM),
        scratch_shapes=[
            pltpu.VMEM((N_DEV - 1, D_MODEL, H_PER * DH), jnp.float32),
            pltpu.VMEM((N_DEV - 1, H_PER * DH, D_MODEL), jnp.float32),
            pltpu.VMEM((SQ, H_PER * DH), jnp.float32),
            pltpu.SemaphoreType.DMA((N_DEV - 1,)),
            pltpu.SemaphoreType.DMA((N_DEV - 1,)),
            pltpu.SemaphoreType.DMA((N_DEV - 1,)),
            pltpu.SemaphoreType.DMA((N_DEV - 1,)),
        ],
        compiler_params=pltpu.CompilerParams(collective_id=0),
    )(x, Wq, k_r, v_r, Wo)


# baseline (device time: 165878 ns/iter reference)
import jax
import jax.numpy as jnp
from jax import lax
from jax.experimental import pallas as pl
from jax.experimental.pallas import tpu as pltpu

N_DEV = 4
B_PER = 2
SQ = 512
SKV = 512
HG = 32
H_PER = 8
DH = 64
D_MODEL = 768
SCALE = 0.125


def kernel(x, Wq, K_ext, V_ext, Wo):
    my = lax.axis_index("i")

    k_my = lax.dynamic_slice(K_ext, (my * B_PER, 0, 0, 0), (B_PER, SKV, HG, DH))
    v_my = lax.dynamic_slice(V_ext, (my * B_PER, 0, 0, 0), (B_PER, SKV, HG, DH))
    k_r = k_my.reshape(B_PER, SKV, N_DEV, H_PER, DH).transpose(2, 0, 3, 1, 4)
    v_r = v_my.reshape(B_PER, SKV, N_DEV, H_PER, DH).transpose(2, 0, 3, 1, 4)

    def body(x_ref, wq_ref, k_ref, v_ref, wo_ref, out_ref,
             wq_comm, wo_comm, ctx_ref, send_q, recv_q, send_o, recv_o):
        my_pos = lax.axis_index("i")
        left = lax.rem(my_pos + N_DEV - 1, N_DEV)
        right = lax.rem(my_pos + 1, N_DEV)

        barrier_sem = pltpu.get_barrier_semaphore()
        for nbr in (left, right):
            pl.semaphore_signal(
                barrier_sem, inc=1,
                device_id=(nbr,), device_id_type=pl.DeviceIdType.MESH,
            )
        pl.semaphore_wait(barrier_sem, 2)

        qi = lax.broadcasted_iota(jnp.int32, (SQ, SKV), 0)
        ki = lax.broadcasted_iota(jnp.int32, (SQ, SKV), 1)
        mask = (jnp.abs(qi - ki) <= 128) | (ki < 32) | (qi < 32)

        def compute_chunk(wq_c, wo_c, origin, first):
            for b in range(B_PER):
                q = lax.dot_general(
                    x_ref[b], wq_c, (((1,), (0,)), ((), ())),
                    preferred_element_type=jnp.float32,
                )
                for h in range(H_PER):
                    qh = q[:, h * DH:(h + 1) * DH]
                    kh = k_ref[origin, b, h]
                    s = lax.dot_general(
                        qh, kh, (((1,), (1,)), ((), ())),
                        preferred_element_type=jnp.float32,
                    ) * SCALE
                    s = jnp.where(mask, s, -1e9)
                    s = s - jnp.max(s, axis=-1, keepdims=True)
                    w = jnp.exp(s)
                    w = w / jnp.sum(w, axis=-1, keepdims=True)
                    vh = v_ref[origin, b, h]
                    ctx_ref[:, h * DH:(h + 1) * DH] = lax.dot_general(
                        w, vh, (((1,), (0,)), ((), ())),
                        preferred_element_type=jnp.float32,
                    )
                partial = lax.dot_general(
                    ctx_ref[...], wo_c, (((1,), (0,)), ((), ())),
                    preferred_element_type=jnp.float32,
                )
                if first:
                    out_ref[b] = partial
                else:
                    out_ref[b] = out_ref[b] + partial

        for h in range(N_DEV - 1):
            src_q = wq_ref if h == 0 else wq_comm.at[h - 1]
            src_o = wo_ref if h == 0 else wo_comm.at[h - 1]
            rdma_q = pltpu.make_async_remote_copy(
                src_ref=src_q, dst_ref=wq_comm.at[h],
                send_sem=send_q.at[h], recv_sem=recv_q.at[h],
                device_id=(right,), device_id_type=pl.DeviceIdType.MESH,
            )
            rdma_o = pltpu.make_async_remote_copy(
                src_ref=src_o, dst_ref=wo_comm.at[h],
                send_sem=send_o.at[h], recv_sem=recv_o.at[h],
                device_id=(right,), device_id_type=pl.DeviceIdType.MESH,
            )
            rdma_q.start()
            rdma_o.start()
            origin = lax.rem(my_pos - h + N_DEV, N_DEV)
            if h == 0:
                compute_chunk(wq_ref[...], wo_ref[...], origin, first=True)
            else:
                compute_chunk(wq_comm[h - 1], wo_comm[h - 1], origin, first=False)
            rdma_q.wait()
            rdma_o.wait()

        origin = lax.rem(my_pos - (N_DEV - 1) + N_DEV, N_DEV)
        compute_chunk(wq_comm[N_DEV - 2], wo_comm[N_DEV - 2], origin, first=False)

    return pl.pallas_call(
        body,
        out_shape=jax.ShapeDtypeStruct((B_PER, SQ, D_MODEL), jnp.float32),
        in_specs=[
            pl.BlockSpec(memory_space=pltpu.VMEM),
            pl.BlockSpec(memory_space=pltpu.VMEM),
            pl.BlockSpec(memory_space=pltpu.VMEM),
            pl.BlockSpec(memory_space=pltpu.VMEM),
            pl.BlockSpec(memory_space=pltpu.VMEM),
        ],
        out_specs=pl.BlockSpec(memory_space=pltpu.VMEM),
        scratch_shapes=[
            pltpu.VMEM((N_DEV - 1, D_MODEL, H_PER * DH), jnp.float32),
            pltpu.VMEM((N_DEV - 1, H_PER * DH, D_MODEL), jnp.float32),
            pltpu.VMEM((SQ, H_PER * DH), jnp.float32),
            pltpu.SemaphoreType.DMA((N_DEV - 1,)),
            pltpu.SemaphoreType.DMA((N_DEV - 1,)),
            pltpu.SemaphoreType.DMA((N_DEV - 1,)),
            pltpu.SemaphoreType.DMA((N_DEV - 1,)),
        ],
        compiler_params=pltpu.CompilerParams(
            collective_id=0,
            vmem_limit_bytes=100 * 1024 * 1024,
        ),
    )(x, Wq, k_r, v_r, Wo)


# device time: 147226 ns/iter; 1.1267x vs baseline; 1.1267x over previous
import jax
import jax.numpy as jnp
from jax import lax
from jax.experimental import pallas as pl
from jax.experimental.pallas import tpu as pltpu

N_DEV = 4
B_PER = 2
SQ = 512
SKV = 512
HG = 32
H_PER = 8
DH = 64
D_MODEL = 768
SCALE = 0.125


def kernel(x, Wq, K_ext, V_ext, Wo):
    my = lax.axis_index("i")

    k_my = lax.dynamic_slice(K_ext, (my * B_PER, 0, 0, 0), (B_PER, SKV, HG, DH))
    v_my = lax.dynamic_slice(V_ext, (my * B_PER, 0, 0, 0), (B_PER, SKV, HG, DH))
    k_r = k_my.reshape(B_PER, SKV, N_DEV, H_PER, DH).transpose(2, 0, 3, 1, 4)
    v_r = v_my.reshape(B_PER, SKV, N_DEV, H_PER, DH).transpose(2, 0, 3, 1, 4)

    def body(x_ref, wq_ref, k_ref, v_ref, wo_ref, out_ref,
             wq_comm, wo_comm, ctx_ref, send_q, recv_q, send_o, recv_o):
        my_pos = lax.axis_index("i")

        barrier_sem = pltpu.get_barrier_semaphore()
        for d in range(1, N_DEV):
            peer = lax.rem(my_pos + d, N_DEV)
            pl.semaphore_signal(
                barrier_sem, inc=1,
                device_id=(peer,), device_id_type=pl.DeviceIdType.MESH,
            )
        pl.semaphore_wait(barrier_sem, N_DEV - 1)

        sends = []
        for d in range(1, N_DEV):
            target = lax.rem(my_pos + d, N_DEV)
            slot = N_DEV - 1 - d
            rq = pltpu.make_async_remote_copy(
                src_ref=wq_ref, dst_ref=wq_comm.at[slot],
                send_sem=send_q.at[d - 1], recv_sem=recv_q.at[slot],
                device_id=(target,), device_id_type=pl.DeviceIdType.MESH,
            )
            ro = pltpu.make_async_remote_copy(
                src_ref=wo_ref, dst_ref=wo_comm.at[slot],
                send_sem=send_o.at[d - 1], recv_sem=recv_o.at[slot],
                device_id=(target,), device_id_type=pl.DeviceIdType.MESH,
            )
            rq.start()
            ro.start()
            sends.append((rq, ro))

        qi = lax.broadcasted_iota(jnp.int32, (SQ, SKV), 0)
        ki = lax.broadcasted_iota(jnp.int32, (SQ, SKV), 1)
        mask = (jnp.abs(qi - ki) <= 128) | (ki < 32) | (qi < 32)
        bias = jnp.where(mask, 0.0, -1e9).astype(jnp.float32)

        def compute_chunk(wq_c, wo_c, origin, first):
            for b in range(B_PER):
                q = lax.dot_general(
                    x_ref[b], wq_c, (((1,), (0,)), ((), ())),
                    preferred_element_type=jnp.float32,
                ) * SCALE
                for h in range(H_PER):
                    qh = q[:, h * DH:(h + 1) * DH]
                    kh = k_ref[origin, b, h]
                    s = lax.dot_general(
                        qh, kh, (((1,), (1,)), ((), ())),
                        preferred_element_type=jnp.float32,
                    )
                    w = jnp.exp(s + bias)
                    wsum = jnp.sum(w, axis=-1, keepdims=True)
                    vh = v_ref[origin, b, h]
                    ctx_h = lax.dot_general(
                        w, vh, (((1,), (0,)), ((), ())),
                        preferred_element_type=jnp.float32,
                    )
                    ctx_ref[:, h * DH:(h + 1) * DH] = ctx_h / wsum
                partial = lax.dot_general(
                    ctx_ref[...], wo_c, (((1,), (0,)), ((), ())),
                    preferred_element_type=jnp.float32,
                )
                if first:
                    out_ref[b] = partial
                else:
                    out_ref[b] = out_ref[b] + partial

        compute_chunk(wq_ref[...], wo_ref[...], my_pos, first=True)

        for slot in (0, 2, 1):
            recv_desc_q = pltpu.make_async_remote_copy(
                src_ref=wq_ref, dst_ref=wq_comm.at[slot],
                send_sem=send_q.at[0], recv_sem=recv_q.at[slot],
                device_id=(my_pos,), device_id_type=pl.DeviceIdType.MESH,
            )
            recv_desc_o = pltpu.make_async_remote_copy(
                src_ref=wo_ref, dst_ref=wo_comm.at[slot],
                send_sem=send_o.at[0], recv_sem=recv_o.at[slot],
                device_id=(my_pos,), device_id_type=pl.DeviceIdType.MESH,
            )
            recv_desc_q.wait_recv()
            recv_desc_o.wait_recv()
            origin = lax.rem(my_pos + slot + 1, N_DEV)
            compute_chunk(wq_comm[slot], wo_comm[slot], origin, first=False)

        for rq, ro in sends:
            rq.wait_send()
            ro.wait_send()

    return pl.pallas_call(
        body,
        out_shape=jax.ShapeDtypeStruct((B_PER, SQ, D_MODEL), jnp.float32),
        in_specs=[
            pl.BlockSpec(memory_space=pltpu.VMEM),
            pl.BlockSpec(memory_space=pltpu.VMEM),
            pl.BlockSpec(memory_space=pltpu.VMEM),
            pl.BlockSpec(memory_space=pltpu.VMEM),
            pl.BlockSpec(memory_space=pltpu.VMEM),
        ],
        out_specs=pl.BlockSpec(memory_space=pltpu.VMEM),
        scratch_shapes=[
            pltpu.VMEM((N_DEV - 1, D_MODEL, H_PER * DH), jnp.float32),
            pltpu.VMEM((N_DEV - 1, H_PER * DH, D_MODEL), jnp.float32),
            pltpu.VMEM((SQ, H_PER * DH), jnp.float32),
            pltpu.SemaphoreType.DMA((N_DEV - 1,)),
            pltpu.SemaphoreType.DMA((N_DEV - 1,)),
            pltpu.SemaphoreType.DMA((N_DEV - 1,)),
            pltpu.SemaphoreType.DMA((N_DEV - 1,)),
        ],
        compiler_params=pltpu.CompilerParams(
            collective_id=0,
            vmem_limit_bytes=100 * 1024 * 1024,
        ),
    )(x, Wq, k_r, v_r, Wo)


# device time: 98344 ns/iter; 1.6867x vs baseline; 1.4971x over previous
import jax
import jax.numpy as jnp
from jax import lax
from jax.experimental import pallas as pl
from jax.experimental.pallas import tpu as pltpu

N_DEV = 4
B_PER = 2
SQ = 512
SKV = 512
HG = 32
H_PER = 8
DH = 64
D_MODEL = 768
SCALE = 0.125


def kernel(x, Wq, K_ext, V_ext, Wo):
    my = lax.axis_index("i")

    k_my = lax.dynamic_slice(K_ext, (my * B_PER, 0, 0, 0), (B_PER, SKV, HG, DH))
    v_my = lax.dynamic_slice(V_ext, (my * B_PER, 0, 0, 0), (B_PER, SKV, HG, DH))
    k_r = k_my.reshape(B_PER, SKV, N_DEV, H_PER, DH).transpose(2, 0, 3, 1, 4)
    v_r = v_my.reshape(B_PER, SKV, N_DEV, H_PER, DH).transpose(2, 0, 3, 1, 4)
    x = x.astype(jnp.bfloat16)
    k_r = k_r.astype(jnp.bfloat16)
    v_r = v_r.astype(jnp.bfloat16)
    Wq = Wq.astype(jnp.bfloat16)
    Wo = Wo.astype(jnp.bfloat16)

    def body(x_ref, wq_ref, k_ref, v_ref, wo_ref, out_ref,
             wq_comm, wo_comm, ctx_ref, send_q, recv_q, send_o, recv_o):
        my_pos = lax.axis_index("i")

        barrier_sem = pltpu.get_barrier_semaphore()
        for d in range(1, N_DEV):
            peer = lax.rem(my_pos + d, N_DEV)
            pl.semaphore_signal(
                barrier_sem, inc=1,
                device_id=(peer,), device_id_type=pl.DeviceIdType.MESH,
            )
        pl.semaphore_wait(barrier_sem, N_DEV - 1)

        sends = []
        for d in range(1, N_DEV):
            target = lax.rem(my_pos + d, N_DEV)
            slot = N_DEV - 1 - d
            rq = pltpu.make_async_remote_copy(
                src_ref=wq_ref, dst_ref=wq_comm.at[slot],
                send_sem=send_q.at[d - 1], recv_sem=recv_q.at[slot],
                device_id=(target,), device_id_type=pl.DeviceIdType.MESH,
            )
            ro = pltpu.make_async_remote_copy(
                src_ref=wo_ref, dst_ref=wo_comm.at[slot],
                send_sem=send_o.at[d - 1], recv_sem=recv_o.at[slot],
                device_id=(target,), device_id_type=pl.DeviceIdType.MESH,
            )
            rq.start()
            ro.start()
            sends.append((rq, ro))

        qi = lax.broadcasted_iota(jnp.int32, (SQ, SKV), 0)
        ki = lax.broadcasted_iota(jnp.int32, (SQ, SKV), 1)
        mask = (jnp.abs(qi - ki) <= 128) | (ki < 32) | (qi < 32)
        bias = jnp.where(mask, 0.0, -1e9).astype(jnp.float32)

        def compute_chunk(wq_c, wo_c, origin, first):
            for b in range(B_PER):
                q = lax.dot_general(
                    x_ref[b], wq_c, (((1,), (0,)), ((), ())),
                    preferred_element_type=jnp.float32,
                )
                q = (q * SCALE).astype(jnp.bfloat16)
                for h in range(H_PER):
                    qh = q[:, h * DH:(h + 1) * DH]
                    kh = k_ref[origin, b, h]
                    s = lax.dot_general(
                        qh, kh, (((1,), (1,)), ((), ())),
                        preferred_element_type=jnp.float32,
                    )
                    w = jnp.exp(s + bias)
                    wsum = jnp.sum(w, axis=-1, keepdims=True)
                    w = w.astype(jnp.bfloat16)
                    vh = v_ref[origin, b, h]
                    ctx_h = lax.dot_general(
                        w, vh, (((1,), (0,)), ((), ())),
                        preferred_element_type=jnp.float32,
                    )
                    ctx_ref[:, h * DH:(h + 1) * DH] = (ctx_h / wsum).astype(jnp.bfloat16)
                partial = lax.dot_general(
                    ctx_ref[...], wo_c, (((1,), (0,)), ((), ())),
                    preferred_element_type=jnp.float32,
                )
                if first:
                    out_ref[b] = partial
                else:
                    out_ref[b] = out_ref[b] + partial

        compute_chunk(wq_ref[...], wo_ref[...], my_pos, first=True)

        for slot in (0, 2, 1):
            recv_desc_q = pltpu.make_async_remote_copy(
                src_ref=wq_ref, dst_ref=wq_comm.at[slot],
                send_sem=send_q.at[0], recv_sem=recv_q.at[slot],
                device_id=(my_pos,), device_id_type=pl.DeviceIdType.MESH,
            )
            recv_desc_o = pltpu.make_async_remote_copy(
                src_ref=wo_ref, dst_ref=wo_comm.at[slot],
                send_sem=send_o.at[0], recv_sem=recv_o.at[slot],
                device_id=(my_pos,), device_id_type=pl.DeviceIdType.MESH,
            )
            recv_desc_q.wait_recv()
            recv_desc_o.wait_recv()
            origin = lax.rem(my_pos + slot + 1, N_DEV)
            compute_chunk(wq_comm[slot], wo_comm[slot], origin, first=False)

        for rq, ro in sends:
            rq.wait_send()
            ro.wait_send()

    return pl.pallas_call(
        body,
        out_shape=jax.ShapeDtypeStruct((B_PER, SQ, D_MODEL), jnp.float32),
        in_specs=[
            pl.BlockSpec(memory_space=pltpu.VMEM),
            pl.BlockSpec(memory_space=pltpu.VMEM),
            pl.BlockSpec(memory_space=pltpu.VMEM),
            pl.BlockSpec(memory_space=pltpu.VMEM),
            pl.BlockSpec(memory_space=pltpu.VMEM),
        ],
        out_specs=pl.BlockSpec(memory_space=pltpu.VMEM),
        scratch_shapes=[
            pltpu.VMEM((N_DEV - 1, D_MODEL, H_PER * DH), jnp.bfloat16),
            pltpu.VMEM((N_DEV - 1, H_PER * DH, D_MODEL), jnp.bfloat16),
            pltpu.VMEM((SQ, H_PER * DH), jnp.bfloat16),
            pltpu.SemaphoreType.DMA((N_DEV - 1,)),
            pltpu.SemaphoreType.DMA((N_DEV - 1,)),
            pltpu.SemaphoreType.DMA((N_DEV - 1,)),
            pltpu.SemaphoreType.DMA((N_DEV - 1,)),
        ],
        compiler_params=pltpu.CompilerParams(
            collective_id=0,
            vmem_limit_bytes=100 * 1024 * 1024,
        ),
    )(x, Wq, k_r, v_r, Wo)


# device time: 89980 ns/iter; 1.8435x vs baseline; 1.0930x over previous
import jax
import jax.numpy as jnp
from jax import lax
from jax.experimental import pallas as pl
from jax.experimental.pallas import tpu as pltpu

N_DEV = 4
B_PER = 2
SQ = 512
SKV = 512
HG = 32
H_PER = 8
DH = 64
D_MODEL = 768
SCALE = 0.125


def kernel(x, Wq, K_ext, V_ext, Wo):
    my = lax.axis_index("i")

    k_my = lax.dynamic_slice(K_ext, (my * B_PER, 0, 0, 0), (B_PER, SKV, HG, DH))
    v_my = lax.dynamic_slice(V_ext, (my * B_PER, 0, 0, 0), (B_PER, SKV, HG, DH))
    k_r = k_my.reshape(B_PER, SKV, N_DEV, H_PER, DH).transpose(2, 0, 3, 1, 4)
    v_r = v_my.reshape(B_PER, SKV, N_DEV, H_PER, DH).transpose(2, 0, 3, 1, 4)
    x = (x * SCALE).astype(jnp.bfloat16)
    k_r = k_r.astype(jnp.bfloat16)
    v_r = v_r.astype(jnp.bfloat16)
    Wq = Wq.astype(jnp.bfloat16)
    Wo = Wo.astype(jnp.bfloat16)

    def body(x_ref, wq_ref, k_ref, v_ref, wo_ref, out_ref,
             wq_comm, wo_comm, ctx_ref, send_q, recv_q, send_o, recv_o):
        my_pos = lax.axis_index("i")

        barrier_sem = pltpu.get_barrier_semaphore()
        for d in range(1, N_DEV):
            peer = lax.rem(my_pos + d, N_DEV)
            pl.semaphore_signal(
                barrier_sem, inc=1,
                device_id=(peer,), device_id_type=pl.DeviceIdType.MESH,
            )
        pl.semaphore_wait(barrier_sem, N_DEV - 1)

        sends = []
        for d in range(1, N_DEV):
            target = lax.rem(my_pos + d, N_DEV)
            slot = N_DEV - 1 - d
            rq = pltpu.make_async_remote_copy(
                src_ref=wq_ref, dst_ref=wq_comm.at[slot],
                send_sem=send_q.at[d - 1], recv_sem=recv_q.at[slot],
                device_id=(target,), device_id_type=pl.DeviceIdType.MESH,
            )
            ro = pltpu.make_async_remote_copy(
                src_ref=wo_ref, dst_ref=wo_comm.at[slot],
                send_sem=send_o.at[d - 1], recv_sem=recv_o.at[slot],
                device_id=(target,), device_id_type=pl.DeviceIdType.MESH,
            )
            rq.start()
            ro.start()
            sends.append((rq, ro))

        qi = lax.broadcasted_iota(jnp.int32, (SQ, SKV), 0)
        ki = lax.broadcasted_iota(jnp.int32, (SQ, SKV), 1)
        mask = (jnp.abs(qi - ki) <= 128) | (ki < 32) | (qi < 32)
        bias = jnp.where(mask, 0.0, -1e9).astype(jnp.bfloat16)

        def compute_chunk(wq_c, wo_c, origin, first):
            for b in range(B_PER):
                q = lax.dot_general(
                    x_ref[b], wq_c, (((1,), (0,)), ((), ())),
                    preferred_element_type=jnp.float32,
                ).astype(jnp.bfloat16)
                for h in range(H_PER):
                    qh = q[:, h * DH:(h + 1) * DH]
                    kh = k_ref[origin, b, h]
                    s = lax.dot_general(
                        qh, kh, (((1,), (1,)), ((), ())),
                        preferred_element_type=jnp.float32,
                    ).astype(jnp.bfloat16)
                    w = jnp.exp(s + bias)
                    wsum = jnp.sum(w, axis=-1, keepdims=True,
                                   dtype=jnp.float32)
                    vh = v_ref[origin, b, h]
                    ctx_h = lax.dot_general(
                        w, vh, (((1,), (0,)), ((), ())),
                        preferred_element_type=jnp.float32,
                    )
                    ctx_ref[:, h * DH:(h + 1) * DH] = (ctx_h / wsum).astype(jnp.bfloat16)
                partial = lax.dot_general(
                    ctx_ref[...], wo_c, (((1,), (0,)), ((), ())),
                    preferred_element_type=jnp.float32,
                )
                if first:
                    out_ref[b] = partial
                else:
                    out_ref[b] = out_ref[b] + partial

        compute_chunk(wq_ref[...], wo_ref[...], my_pos, first=True)

        for slot in (2, 0, 1):
            recv_desc_q = pltpu.make_async_remote_copy(
                src_ref=wq_ref, dst_ref=wq_comm.at[slot],
                send_sem=send_q.at[0], recv_sem=recv_q.at[slot],
                device_id=(my_pos,), device_id_type=pl.DeviceIdType.MESH,
            )
            recv_desc_o = pltpu.make_async_remote_copy(
                src_ref=wo_ref, dst_ref=wo_comm.at[slot],
                send_sem=send_o.at[0], recv_sem=recv_o.at[slot],
                device_id=(my_pos,), device_id_type=pl.DeviceIdType.MESH,
            )
            recv_desc_q.wait_recv()
            recv_desc_o.wait_recv()
            origin = lax.rem(my_pos + slot + 1, N_DEV)
            compute_chunk(wq_comm[slot], wo_comm[slot], origin, first=False)

        for rq, ro in sends:
            rq.wait_send()
            ro.wait_send()

    return pl.pallas_call(
        body,
        out_shape=jax.ShapeDtypeStruct((B_PER, SQ, D_MODEL), jnp.float32),
        in_specs=[
            pl.BlockSpec(memory_space=pltpu.VMEM),
            pl.BlockSpec(memory_space=pltpu.VMEM),
            pl.BlockSpec(memory_space=pltpu.VMEM),
            pl.BlockSpec(memory_space=pltpu.VMEM),
            pl.BlockSpec(memory_space=pltpu.VMEM),
        ],
        out_specs=pl.BlockSpec(memory_space=pltpu.VMEM),
        scratch_shapes=[
            pltpu.VMEM((N_DEV - 1, D_MODEL, H_PER * DH), jnp.bfloat16),
            pltpu.VMEM((N_DEV - 1, H_PER * DH, D_MODEL), jnp.bfloat16),
            pltpu.VMEM((SQ, H_PER * DH), jnp.bfloat16),
            pltpu.SemaphoreType.DMA((N_DEV - 1,)),
            pltpu.SemaphoreType.DMA((N_DEV - 1,)),
            pltpu.SemaphoreType.DMA((N_DEV - 1,)),
            pltpu.SemaphoreType.DMA((N_DEV - 1,)),
        ],
        compiler_params=pltpu.CompilerParams(
            collective_id=0,
            vmem_limit_bytes=100 * 1024 * 1024,
        ),
    )(x, Wq, k_r, v_r, Wo)
